# baseline (device time: 62042 ns/iter reference)
import jax
import jax.numpy as jnp
from jax import lax
from jax.experimental import pallas as pl
from jax.experimental.pallas import tpu as pltpu

N_DEV = 32
M_PER = 32


def kernel(x, w_mat):
    k_loc, n = w_mat.shape
    m = x.shape[0]

    def body(x_ref, w_ref, out_ref, acc_ref, inbox_ref, send_sems, recv_sems):
        my = lax.axis_index("i")

        bar = pltpu.get_barrier_semaphore()
        for off in range(1, N_DEV):
            pl.semaphore_signal(
                bar, inc=1,
                device_id=(lax.rem(my + off, N_DEV),),
                device_id_type=pl.DeviceIdType.MESH,
            )
        pl.semaphore_wait(bar, N_DEV - 1)

        acc_ref[...] = jnp.dot(
            x_ref[...], w_ref[...], preferred_element_type=jnp.float32
        )

        rdmas = []
        for j in range(1, N_DEV):
            tgt = lax.rem(my + j, N_DEV)
            rdma = pltpu.make_async_remote_copy(
                src_ref=acc_ref.at[pl.ds(tgt * M_PER, M_PER), :],
                dst_ref=inbox_ref.at[j - 1],
                send_sem=send_sems.at[j - 1],
                recv_sem=recv_sems.at[j - 1],
                device_id=(tgt,),
                device_id_type=pl.DeviceIdType.MESH,
            )
            rdma.start()
            rdmas.append(rdma)

        result = acc_ref[pl.ds(my * M_PER, M_PER), :]
        for j in range(1, N_DEV):
            rdmas[j - 1].wait_recv()
            result = result + inbox_ref[j - 1]
        out_ref[...] = jnp.maximum(result, 0.0)

        for j in range(1, N_DEV):
            rdmas[j - 1].wait_send()

    return pl.pallas_call(
        body,
        out_shape=jax.ShapeDtypeStruct((M_PER, n), jnp.float32),
        in_specs=[
            pl.BlockSpec(memory_space=pltpu.VMEM),
            pl.BlockSpec(memory_space=pltpu.VMEM),
        ],
        out_specs=pl.BlockSpec(memory_space=pltpu.VMEM),
        scratch_shapes=[
            pltpu.VMEM((m, n), jnp.float32),
            pltpu.VMEM((N_DEV - 1, M_PER, n), jnp.float32),
            pltpu.SemaphoreType.DMA((N_DEV - 1,)),
            pltpu.SemaphoreType.DMA((N_DEV - 1,)),
        ],
        compiler_params=pltpu.CompilerParams(collective_id=0),
    )(x, w_mat)
